# baseline (speedup 1.0000x reference)
import functools

import jax
import jax.numpy as jnp
from jax import lax
from jax.experimental import pallas as pl
from jax.experimental.pallas import tpu as pltpu
from jax.experimental.pallas import tpu_sc as plsc

_SEQ = 8192
_DIM = 1024
_NC = 2
_NS = 16
_NW = _NC * _NS
_ROWS_PER_W = _SEQ // _NW
_CHUNK = 32
_NSTEP = _ROWS_PER_W // _CHUNK
_NBUF = 2


def _copy_body(table_hbm, out_hbm, tbuf, sbuf, in_sems, out_sems):
    cid = lax.axis_index("c")
    sid = lax.axis_index("s")
    wid = sid * _NC + cid
    base = wid * _ROWS_PER_W

    def stage(i):
        if i % 2 == 0:
            return tbuf.at[(i // 2) % _NBUF]
        return sbuf.at[sid, (i // 2) % _NBUF]

    def read(i):
        return pltpu.make_async_copy(
            table_hbm.at[pl.ds(base + i * _CHUNK, _CHUNK)],
            stage(i),
            in_sems.at[i % (2 * _NBUF)],
        )

    def write(i):
        return pltpu.make_async_copy(
            stage(i),
            out_hbm.at[0, pl.ds(base + i * _CHUNK, _CHUNK)],
            out_sems.at[i % (2 * _NBUF)],
        )

    depth = 2 * _NBUF
    waited = [False] * _NSTEP
    for i in range(min(depth, _NSTEP)):
        read(i).start()
    for i in range(_NSTEP):
        read(i).wait()
        write(i).start()
        j = i + depth
        if j < _NSTEP:
            write(i).wait()
            waited[i] = True
            read(j).start()
    for i in range(_NSTEP):
        if not waited[i]:
            write(i).wait()


@jax.jit
def _positional_copy(emb_weight):
    mesh = plsc.VectorSubcoreMesh(core_axis_name="c", subcore_axis_name="s")
    k = functools.partial(
        pl.kernel,
        mesh=mesh,
        out_type=jax.ShapeDtypeStruct((1, _SEQ, _DIM), jnp.float32),
        scratch_types=[
            pltpu.VMEM((_NBUF, _CHUNK, _DIM), jnp.float32),
            pltpu.VMEM_SHARED((_NS, _NBUF, _CHUNK, _DIM), jnp.float32),
            pltpu.SemaphoreType.DMA((2 * _NBUF,)),
            pltpu.SemaphoreType.DMA((2 * _NBUF,)),
        ],
    )(_copy_body)
    return k(emb_weight)


def kernel(x, emb_weight):
    del x
    return _positional_copy(emb_weight)


# speedup vs baseline: 1.0058x; 1.0058x over previous
import functools

import jax
import jax.numpy as jnp
from jax import lax
from jax.experimental import pallas as pl
from jax.experimental.pallas import tpu as pltpu
from jax.experimental.pallas import tpu_sc as plsc

_SEQ = 8192
_DIM = 1024
_NC = 2
_NS = 16
_NW = _NC * _NS
_ROWS_PER_W = _SEQ // _NW
_CHUNK = 16
_NSTEP = _ROWS_PER_W // _CHUNK
_NBUF = 7
_LAZY = 3


def _copy_body(table_hbm, out_hbm, buf, in_sems, out_sems):
    wid = lax.axis_index("s") * _NC + lax.axis_index("c")
    base = wid * _ROWS_PER_W

    def read(i):
        return pltpu.make_async_copy(
            table_hbm.at[pl.ds(base + i * _CHUNK, _CHUNK)],
            buf.at[i % _NBUF],
            in_sems.at[i % _NBUF],
        )

    def write(i):
        return pltpu.make_async_copy(
            buf.at[i % _NBUF],
            out_hbm.at[0, pl.ds(base + i * _CHUNK, _CHUNK)],
            out_sems.at[i % _NBUF],
        )

    waited = [False] * _NSTEP
    for i in range(min(_NBUF, _NSTEP)):
        read(i).start()
    for i in range(_NSTEP):
        read(i).wait()
        write(i).start()
        j = i - _LAZY
        if j >= 0 and j + _NBUF < _NSTEP:
            write(j).wait()
            waited[j] = True
            read(j + _NBUF).start()
    for i in range(_NSTEP):
        if not waited[i]:
            write(i).wait()


@jax.jit
def _positional_copy(emb_weight):
    mesh = plsc.VectorSubcoreMesh(core_axis_name="c", subcore_axis_name="s")
    k = functools.partial(
        pl.kernel,
        mesh=mesh,
        out_type=jax.ShapeDtypeStruct((1, _SEQ, _DIM), jnp.float32),
        scratch_types=[
            pltpu.VMEM((_NBUF, _CHUNK, _DIM), jnp.float32),
            pltpu.SemaphoreType.DMA((_NBUF,)),
            pltpu.SemaphoreType.DMA((_NBUF,)),
        ],
    )(_copy_body)
    return k(emb_weight)


def kernel(x, emb_weight):
    del x
    return _positional_copy(emb_weight)
